# baseline (speedup 1.0000x reference)
import jax
import jax.numpy as jnp
from jax.experimental import pallas as pl
from jax.experimental.pallas import tpu as pltpu

_ROWS_PER_BLOCK = 2000
_COPY_CHUNKS = 8


def _fused_kernel(user_in, x_ref, w_ref, b_ref, user_out, o_ref, sem):
    i = pl.program_id(0)
    chunk_rows = user_in.shape[0] // _COPY_CHUNKS

    @pl.when(i == 0)
    def _start_copies():
        for c in range(_COPY_CHUNKS):
            pltpu.make_async_copy(
                user_in.at[pl.ds(c * chunk_rows, chunk_rows), :],
                user_out.at[pl.ds(c * chunk_rows, chunk_rows), :],
                sem.at[c],
            ).start()

    o_ref[...] = (
        jnp.dot(x_ref[...], w_ref[...], preferred_element_type=jnp.float32)
        + b_ref[...]
    )

    @pl.when(i == pl.num_programs(0) - 1)
    def _wait_copies():
        for c in range(_COPY_CHUNKS):
            pltpu.make_async_copy(
                user_in.at[pl.ds(c * chunk_rows, chunk_rows), :],
                user_out.at[pl.ds(c * chunk_rows, chunk_rows), :],
                sem.at[c],
            ).wait()


def kernel(movie_x, user_emb_weight, W, b):
    m, k = movie_x.shape
    n = W.shape[1]
    users, d = user_emb_weight.shape
    user_out, movie = pl.pallas_call(
        _fused_kernel,
        grid=(pl.cdiv(m, _ROWS_PER_BLOCK),),
        in_specs=[
            pl.BlockSpec(memory_space=pltpu.MemorySpace.HBM),
            pl.BlockSpec((_ROWS_PER_BLOCK, k), lambda i: (i, 0)),
            pl.BlockSpec((k, n), lambda i: (0, 0)),
            pl.BlockSpec((n,), lambda i: (0,)),
        ],
        out_specs=[
            pl.BlockSpec(memory_space=pltpu.MemorySpace.HBM),
            pl.BlockSpec((_ROWS_PER_BLOCK, n), lambda i: (i, 0)),
        ],
        out_shape=[
            jax.ShapeDtypeStruct((users, d), jnp.float32),
            jax.ShapeDtypeStruct((m, n), jnp.float32),
        ],
        scratch_shapes=[pltpu.SemaphoreType.DMA((_COPY_CHUNKS,))],
    )(user_emb_weight, movie_x, W, b)
    return (user_out, movie)


# speedup vs baseline: 15.1472x; 15.1472x over previous
import jax
import jax.numpy as jnp
from jax.experimental import pallas as pl

_GRID = 50
_MOVIE_ROWS = 2000
_USER_ROWS = 20000


def _fused_kernel(u_ref, x_ref, w_ref, b_ref, uo_ref, o_ref):
    uo_ref[...] = u_ref[...]
    o_ref[...] = (
        jnp.dot(x_ref[...], w_ref[...], preferred_element_type=jnp.float32)
        + b_ref[...]
    )


def kernel(movie_x, user_emb_weight, W, b):
    m, k = movie_x.shape
    n = W.shape[1]
    users, d = user_emb_weight.shape
    user_out, movie = pl.pallas_call(
        _fused_kernel,
        grid=(_GRID,),
        in_specs=[
            pl.BlockSpec((_USER_ROWS, d), lambda i: (i, 0)),
            pl.BlockSpec((_MOVIE_ROWS, k), lambda i: (i, 0)),
            pl.BlockSpec((k, n), lambda i: (0, 0)),
            pl.BlockSpec((n,), lambda i: (0,)),
        ],
        out_specs=[
            pl.BlockSpec((_USER_ROWS, d), lambda i: (i, 0)),
            pl.BlockSpec((_MOVIE_ROWS, n), lambda i: (i, 0)),
        ],
        out_shape=[
            jax.ShapeDtypeStruct((users, d), jnp.float32),
            jax.ShapeDtypeStruct((m, n), jnp.float32),
        ],
    )(user_emb_weight, movie_x, W, b)
    return (user_out, movie)
